# Initial kernel scaffold; baseline (speedup 1.0000x reference)
import jax
import jax.numpy as jnp
from jax.experimental import pallas as pl


def kernel(x, edge_index, batch, W_e0, a_src_e0, a_dst_e0, b_e0, W_e1, a_src_e1, a_dst_e1, b_e1, Wg1, bg1, Wg2, bg2, W_d0, a_src_d0, a_dst_d0, b_d0, W_d1, a_src_d1, a_dst_d1, b_d1):
    raise NotImplementedError("write your pallas kernel here")


# speedup vs baseline: 10.9993x; 10.9993x over previous
import functools

import jax
import jax.numpy as jnp
from jax import lax
from jax.experimental import pallas as pl
from jax.experimental.pallas import tpu as pltpu
from jax.experimental.pallas import tpu_sc as plsc

N = 10000
E = 320000
G = 16
NC = 2
NS = 16
NW = NC * NS
EPT = E // NW
C = 16
NCH = EPT // C
WTN = 10
SEG = N // WTN
CHR = 40
LW = 128
BCOL = 120
CNT0 = 16

_mesh = plsc.VectorSubcoreMesh(
    core_axis_name="c", subcore_axis_name="s", num_cores=NC, num_subcores=NS)
_sc_params = pltpu.CompilerParams(needs_layout_passes=False)

_f32 = jnp.float32


def _sds(shape, dtype=_f32):
    return jax.ShapeDtypeStruct(shape, dtype)


def _sc_att_z(srcs, dsts, stab, heads, sd_off, with_cnt):
    @functools.partial(
        pl.kernel,
        out_type=(_sds((NW, NCH, LW)), _sds((NC, N, LW))),
        mesh=_mesh,
        compiler_params=_sc_params,
        scratch_types=[
            pltpu.VMEM((EPT,), jnp.int32),
            pltpu.VMEM((EPT,), jnp.int32),
            pltpu.VMEM((C, LW), _f32),
            pltpu.VMEM((C, LW), _f32),
            pltpu.VMEM((C, LW), _f32),
            pltpu.VMEM((LW,), _f32),
            pltpu.VMEM((CHR, LW), _f32),
            pltpu.VMEM_SHARED((N, LW), _f32),
        ],
    )
    def k(srcs_h, dsts_h, stab_h, zeros_h, pbuf_h, zparts_h,
          srcbuf, dstbuf, ssbuf, sdbuf, pchunk, pchunk_t, wb, z_sh):
        c = lax.axis_index("c")
        s = lax.axis_index("s")
        w = s * NC + c
        pltpu.sync_copy(srcs_h.at[w], srcbuf)
        pltpu.sync_copy(dsts_h.at[w], dstbuf)

        @pl.when(s < WTN)
        def _zero():
            pltpu.sync_copy(zeros_h, wb)
            for i in range(SEG // CHR):
                pltpu.sync_copy(wb, z_sh.at[pl.ds(s * SEG + i * CHR, CHR)])

        plsc.subcore_barrier()
        iota = lax.iota(jnp.int32, 16)
        zero16 = jnp.zeros((16,), _f32)
        one16 = jnp.ones((16,), _f32)
        for i in range(C):
            pchunk[i, pl.ds(0, 16)] = zero16
            if with_cnt:
                pchunk[i, pl.ds(CNT0, 16)] = zero16

        def chunk(j, carry):
            srcv = srcbuf[pl.ds(j * C, C)]
            dstv = dstbuf[pl.ds(j * C, C)]
            pltpu.sync_copy(stab_h.at[srcv], ssbuf)
            pltpu.sync_copy(stab_h.at[dstv], sdbuf)
            for h in range(heads):
                sv = (plsc.load_gather(ssbuf, [iota, jnp.full((16,), h, jnp.int32)])
                      + plsc.load_gather(sdbuf, [iota, jnp.full((16,), sd_off + h, jnp.int32)]))
                ev = jnp.where(sv > 0, sv, 0.2 * sv)
                pv = jnp.exp(ev)
                plsc.store_scatter(pchunk, [iota, jnp.full((16,), h, jnp.int32)], pv)
                pchunk_t[pl.ds(h * 16, 16)] = pv
            if with_cnt:
                bsrc = plsc.load_gather(ssbuf, [iota, jnp.full((16,), BCOL, jnp.int32)])
                bcol = CNT0 + bsrc.astype(jnp.int32)
                plsc.store_scatter(pchunk, [iota, bcol], one16)
            pltpu.sync_copy(pchunk_t, pbuf_h.at[w, j])
            pltpu.sync_copy(pchunk, z_sh.at[dstv], add=True)
            if with_cnt:
                plsc.store_scatter(pchunk, [iota, bcol], zero16)
            return carry

        lax.fori_loop(0, NCH, chunk, 0)
        plsc.subcore_barrier()

        @pl.when(s < WTN)
        def _wb():
            for i in range(SEG // CHR):
                pltpu.sync_copy(z_sh.at[pl.ds(s * SEG + i * CHR, CHR)], wb)
                pltpu.sync_copy(wb, zparts_h.at[c, pl.ds(s * SEG + i * CHR, CHR)])

    zeros = jnp.zeros((CHR, LW), _f32)
    return k(srcs, dsts, stab, zeros)


def _sc_att_agg(srcs, dsts, hmat, pbuf, rw, heads, oc):
    hw = heads * oc

    @functools.partial(
        pl.kernel,
        out_type=_sds((NC, N, LW)),
        mesh=_mesh,
        compiler_params=_sc_params,
        scratch_types=[
            pltpu.VMEM((EPT,), jnp.int32),
            pltpu.VMEM((EPT,), jnp.int32),
            pltpu.VMEM((LW,), _f32),
            pltpu.VMEM((C, LW), _f32),
            pltpu.VMEM((C, hw), _f32),
            pltpu.VMEM((C, LW), _f32),
            pltpu.VMEM((CHR, LW), _f32),
            pltpu.VMEM_SHARED((N, LW), _f32),
        ],
    )
    def k(srcs_h, dsts_h, hmat_h, pbuf_h, rw_h, zeros_h, oparts_h,
          srcbuf, dstbuf, pchunk_t, rbuf, hbuf, msgbuf, wb, o_sh):
        c = lax.axis_index("c")
        s = lax.axis_index("s")
        w = s * NC + c
        pltpu.sync_copy(srcs_h.at[w], srcbuf)
        pltpu.sync_copy(dsts_h.at[w], dstbuf)

        @pl.when(s < WTN)
        def _zero():
            pltpu.sync_copy(zeros_h, wb)
            for i in range(SEG // CHR):
                pltpu.sync_copy(wb, o_sh.at[pl.ds(s * SEG + i * CHR, CHR)])

        plsc.subcore_barrier()
        iota = lax.iota(jnp.int32, 16)

        def chunk(j, carry):
            srcv = srcbuf[pl.ds(j * C, C)]
            dstv = dstbuf[pl.ds(j * C, C)]
            pltpu.sync_copy(pbuf_h.at[w, j], pchunk_t)
            pltpu.sync_copy(rw_h.at[dstv], rbuf)
            pltpu.sync_copy(hmat_h.at[srcv], hbuf)
            wv = [pchunk_t[pl.ds(h * 16, 16)]
                  * plsc.load_gather(rbuf, [iota, jnp.full((16,), h, jnp.int32)])
                  for h in range(heads)]

            def edge(e, ecarry):
                eidx = jnp.full((16,), e, jnp.int32)
                ws = [wv[h].at[eidx].get(mode="promise_in_bounds")
                      for h in range(heads)]
                for kk in range(oc // 16):
                    acc = ws[0] * hbuf[e, pl.ds(kk * 16, 16)]
                    for h in range(1, heads):
                        acc = acc + ws[h] * hbuf[e, pl.ds(h * oc + kk * 16, 16)]
                    msgbuf[e, pl.ds(kk * 16, 16)] = acc
                return ecarry

            lax.fori_loop(0, C, edge, 0)
            pltpu.sync_copy(msgbuf, o_sh.at[dstv], add=True)
            return carry

        lax.fori_loop(0, NCH, chunk, 0)
        plsc.subcore_barrier()

        @pl.when(s < WTN)
        def _wb():
            for i in range(SEG // CHR):
                pltpu.sync_copy(o_sh.at[pl.ds(s * SEG + i * CHR, CHR)], wb)
                pltpu.sync_copy(wb, oparts_h.at[c, pl.ds(s * SEG + i * CHR, CHR)])

    zeros = jnp.zeros((CHR, LW), _f32)
    return k(srcs, dsts, hmat, pbuf, rw, zeros)


_BLK = 1000


def _full(shape):
    return pl.BlockSpec(shape, lambda j: (0,) * len(shape))


def _rows(cols):
    return pl.BlockSpec((_BLK, cols), lambda j: (j, 0))


def _tc_mm_proj(x, W, A, batch2d):
    din = x.shape[1]
    dh = W.shape[1]

    def body(x_r, w_r, a_r, bt_r, h_r, s_r):
        h = jnp.dot(x_r[...], w_r[...], preferred_element_type=_f32)
        h_r[...] = h
        ss = jnp.dot(h, a_r[...], preferred_element_type=_f32)
        iot = lax.broadcasted_iota(jnp.int32, (1, LW), 1)
        bt = bt_r[...].astype(_f32)
        s_r[...] = jnp.where(iot == BCOL, bt, ss)

    return pl.pallas_call(
        body,
        grid=(N // _BLK,),
        in_specs=[_rows(din), _full((din, dh)), _full((dh, LW)),
                  pl.BlockSpec((_BLK, 1), lambda j: (j, 0))],
        out_specs=[_rows(dh), _rows(LW)],
        out_shape=[_sds((N, dh)), _sds((N, LW))],
    )(x, W, A, batch2d)


def _tc_rinv(z0, z1, S, heads):
    def body(z0_r, z1_r, s_r, rw_r):
        ss = s_r[:, :heads]
        sd = s_r[:, heads:2 * heads]
        e = ss + sd
        p = jnp.exp(jnp.where(e > 0, e, 0.2 * e))
        zt = z0_r[:, :heads] + z1_r[:, :heads] + p + 1e-16
        r = 1.0 / (zt * heads)
        pad = jnp.zeros((_BLK, LW - 2 * heads), _f32)
        rw_r[...] = jnp.concatenate([r, p * r, pad], axis=1)

    return pl.pallas_call(
        body,
        grid=(N // _BLK,),
        in_specs=[_rows(LW), _rows(LW), _rows(LW)],
        out_specs=_rows(LW),
        out_shape=_sds((N, LW)),
    )(z0, z1, S)


def _tc_post_pre(o0, o1, rw, hprev, b, Wn, An, heads, oc, relu, batch2d):
    dn = Wn.shape[1]
    hw = heads * oc

    def body(o0_r, o1_r, rw_r, hp_r, b_r, w_r, a_r, bt_r, h_r, s_r):
        o = o0_r[:, :oc] + o1_r[:, :oc]
        hp = hp_r[...]
        wl = rw_r[:, heads:2 * heads]
        for h in range(heads):
            o = o + wl[:, h:h + 1] * hp[:, h * oc:(h + 1) * oc]
        x1 = o + b_r[...]
        if relu:
            x1 = jnp.maximum(x1, 0.0)
        hh = jnp.dot(x1, w_r[...], preferred_element_type=_f32)
        h_r[...] = hh
        ss = jnp.dot(hh, a_r[...], preferred_element_type=_f32)
        iot = lax.broadcasted_iota(jnp.int32, (1, LW), 1)
        bt = bt_r[...].astype(_f32)
        s_r[...] = jnp.where(iot == BCOL, bt, ss)

    return pl.pallas_call(
        body,
        grid=(N // _BLK,),
        in_specs=[_rows(LW), _rows(LW), _rows(LW), _rows(hw),
                  _full((1, oc)), _full((oc, dn)), _full((dn, LW)),
                  pl.BlockSpec((_BLK, 1), lambda j: (j, 0))],
        out_specs=[_rows(dn), _rows(LW)],
        out_shape=[_sds((N, dn)), _sds((N, LW))],
    )(o0, o1, rw, hprev, b, Wn, An, batch2d)


def _tc_pool(o0, o1, rw, hprev, b, batch2d, Wg1, bg1, Wg2, bg2, W_d0, avs, avd,
             heads, oc):
    ng = N // _BLK
    hw = heads * oc
    d0 = W_d0.shape[1]

    def body(o0_r, o1_r, rw_r, hp_r, b_r, bt_r, wg1_r, bg1_r, wg2_r, bg2_r,
             wd0_r, avs_r, avd_r, t_r, tsrow_r, tscol_r, tdcol_r, sg, zg):
        j = pl.program_id(0)
        o = o0_r[:, :oc] + o1_r[:, :oc]
        hp = hp_r[...]
        wl = rw_r[:, heads:2 * heads]
        for h in range(heads):
            o = o + wl[:, h:h + 1] * hp[:, h * oc:(h + 1) * oc]
        hn = o + b_r[...]
        g = jnp.maximum(jnp.dot(hn, wg1_r[...], preferred_element_type=_f32)
                        + bg1_r[...], 0.0)
        g = jnp.dot(g, wg2_r[...], preferred_element_type=_f32) + bg2_r[...]
        p = jnp.exp(g)
        iot = lax.broadcasted_iota(jnp.int32, (1, G), 1)
        mask = (bt_r[...] == iot).astype(_f32)
        mp = mask * p

        @pl.when(j == 0)
        def _():
            sg[...] = jnp.zeros_like(sg)
            zg[...] = jnp.zeros_like(zg)

        dnm = (((0,), (0,)), ((), ()))
        sg[...] += lax.dot_general(mp, hn, dnm, preferred_element_type=_f32)
        zg[...] += lax.dot_general(mp, jnp.ones((_BLK, 1), _f32), dnm,
                                   preferred_element_type=_f32)
        pooled = sg[...] / (zg[...] + 1e-16)
        t = jnp.dot(pooled, wd0_r[...], preferred_element_type=_f32)
        t_r[...] = t
        tscol_r[...] = jnp.dot(t, avs_r[...], preferred_element_type=_f32)
        tdcol_r[...] = jnp.dot(t, avd_r[...], preferred_element_type=_f32)
        tsrow_r[...] = lax.dot_general(avs_r[...], t, (((0,), (1,)), ((), ())),
                                       preferred_element_type=_f32)

    return pl.pallas_call(
        body,
        grid=(ng,),
        in_specs=[_rows(LW), _rows(LW), _rows(LW), _rows(hw),
                  _full((1, oc)), pl.BlockSpec((_BLK, 1), lambda j: (j, 0)),
                  _full(Wg1.shape), _full((1, oc)), _full(Wg2.shape),
                  _full((1, 1)), _full(W_d0.shape), _full((d0, 1)),
                  _full((d0, 1))],
        out_specs=[_full((G, d0)), _full((1, G)), _full((G, 1)), _full((G, 1))],
        out_shape=[_sds((G, d0)), _sds((1, G)), _sds((G, 1)), _sds((G, 1))],
        scratch_shapes=[pltpu.VMEM((G, oc), _f32), pltpu.VMEM((G, 1), _f32)],
    )(o0, o1, rw, hprev, b, batch2d, Wg1, bg1, Wg2, bg2, W_d0, avs, avd)


def _tc_d0(zp0, zp1, batch2d, T, tsrow, tscol, tdcol, b, Wn, An):
    d0 = T.shape[1]
    dn = Wn.shape[1]

    def body(z0_r, z1_r, bt_r, t_r, tsr_r, tsc_r, tdc_r, b_r, w_r, a_r,
             h_r, s_r):
        iot = lax.broadcasted_iota(jnp.int32, (1, G), 1)
        onehot = (bt_r[...] == iot).astype(_f32)
        cnt = z0_r[:, CNT0:CNT0 + G] + z1_r[:, CNT0:CNT0 + G]
        td_n = jnp.dot(onehot, tdc_r[...], preferred_element_type=_f32)
        ts_n = jnp.dot(onehot, tsc_r[...], preferred_element_type=_f32)
        em = tsr_r[...] + td_n
        m = jnp.exp(jnp.where(em > 0, em, 0.2 * em))
        ed = ts_n + td_n
        pd = jnp.exp(jnp.where(ed > 0, ed, 0.2 * ed))
        cm = cnt * m + onehot * pd
        z = jnp.dot(cm, jnp.ones((G, 1), _f32), preferred_element_type=_f32)
        o = jnp.dot(cm, t_r[...], preferred_element_type=_f32) / (z + 1e-16)
        h2 = jnp.maximum(o + b_r[...], 0.0)
        hh = jnp.dot(h2, w_r[...], preferred_element_type=_f32)
        h_r[...] = hh
        s_r[...] = jnp.dot(hh, a_r[...], preferred_element_type=_f32)

    return pl.pallas_call(
        body,
        grid=(N // _BLK,),
        in_specs=[_rows(LW), _rows(LW), pl.BlockSpec((_BLK, 1), lambda j: (j, 0)),
                  _full((G, d0)), _full((1, G)), _full((G, 1)), _full((G, 1)),
                  _full((1, d0)), _full((d0, dn)), _full((dn, LW))],
        out_specs=[_rows(dn), _rows(LW)],
        out_shape=[_sds((N, dn)), _sds((N, LW))],
    )(zp0, zp1, batch2d, T, tsrow, tscol, tdcol, b, Wn, An)


def _tc_final(o0, o1, rw, hmat, b, oc):
    def body(o0_r, o1_r, rw_r, h_r, b_r, out_r):
        wl = rw_r[:, 1:2]
        out_r[...] = o0_r[:, :oc] + o1_r[:, :oc] + wl * h_r[...] + b_r[...]

    return pl.pallas_call(
        body,
        grid=(N // _BLK,),
        in_specs=[_rows(LW), _rows(LW), _rows(LW), _rows(oc), _full((1, oc))],
        out_specs=_rows(oc),
        out_shape=_sds((N, oc)),
    )(o0, o1, rw, hmat, b)


def _blockdiag(a_s, a_d):
    heads, oc = a_s.shape[1], a_s.shape[2]
    eye = jnp.eye(heads, dtype=_f32)
    bs = (a_s[0][:, :, None] * eye[:, None, :]).reshape(heads * oc, heads)
    bd = (a_d[0][:, :, None] * eye[:, None, :]).reshape(heads * oc, heads)
    pad = jnp.zeros((heads * oc, LW - 2 * heads), _f32)
    return jnp.concatenate([bs, bd, pad], axis=1)


def kernel(x, edge_index, batch, W_e0, a_src_e0, a_dst_e0, b_e0, W_e1,
           a_src_e1, a_dst_e1, b_e1, Wg1, bg1, Wg2, bg2, W_d0, a_src_d0,
           a_dst_d0, b_d0, W_d1, a_src_d1, a_dst_d1, b_d1):
    src = edge_index[0].astype(jnp.int32).reshape(NW, EPT)
    dst = edge_index[1].astype(jnp.int32).reshape(NW, EPT)
    batch2d = batch.astype(jnp.int32).reshape(N, 1)

    A0 = _blockdiag(a_src_e0, a_dst_e0)
    A1 = _blockdiag(a_src_e1, a_dst_e1)
    A3 = _blockdiag(a_src_d1, a_dst_d1)
    avs_d0 = a_src_d0[0, 0, :].reshape(-1, 1)
    avd_d0 = a_dst_d0[0, 0, :].reshape(-1, 1)

    H0, S0 = _tc_mm_proj(x, W_e0, A0, batch2d)
    pbuf0, zp0 = _sc_att_z(src, dst, S0, 8, 8, with_cnt=True)
    RW0 = _tc_rinv(zp0[0], zp0[1], S0, 8)
    op0 = _sc_att_agg(src, dst, H0, pbuf0, RW0, 8, 128)
    H1, S1 = _tc_post_pre(op0[0], op0[1], RW0, H0, b_e0.reshape(1, -1),
                          W_e1, A1, 8, 128, True, batch2d)

    pbuf1, zp1 = _sc_att_z(src, dst, S1, 8, 8, with_cnt=False)
    RW1 = _tc_rinv(zp1[0], zp1[1], S1, 8)
    op1 = _sc_att_agg(src, dst, H1, pbuf1, RW1, 8, 64)

    T, tsrow, tscol, tdcol = _tc_pool(
        op1[0], op1[1], RW1, H1, b_e1.reshape(1, -1), batch2d, Wg1,
        bg1.reshape(1, -1), Wg2, bg2.reshape(1, -1), W_d0, avs_d0, avd_d0,
        8, 64)
    H3, S3 = _tc_d0(zp0[0], zp0[1], batch2d, T, tsrow, tscol, tdcol,
                    b_d0.reshape(1, -1), W_d1, A3)

    pbuf3, zp3 = _sc_att_z(src, dst, S3, 1, 1, with_cnt=False)
    RW3 = _tc_rinv(zp3[0], zp3[1], S3, 1)
    op3 = _sc_att_agg(src, dst, H3, pbuf3, RW3, 1, 128)
    out = _tc_final(op3[0], op3[1], RW3, H3, b_d1.reshape(1, -1), 128)
    return out


# speedup vs baseline: 30.8596x; 2.8056x over previous
import functools

import jax
import jax.numpy as jnp
from jax import lax
from jax.experimental import pallas as pl
from jax.experimental.pallas import tpu as pltpu
from jax.experimental.pallas import tpu_sc as plsc

N = 10000
E = 320000
G = 16
NC = 2
NS = 16
NW = NC * NS
EPT = E // NW
C = 16
NCH = EPT // C
WTN = 10
SEG = N // WTN
CHR = 40
LW = 128
BCOL = 120
CNT0 = 16

_mesh = plsc.VectorSubcoreMesh(
    core_axis_name="c", subcore_axis_name="s", num_cores=NC, num_subcores=NS)
_sc_params = pltpu.CompilerParams(needs_layout_passes=False)

_f32 = jnp.float32


def _sds(shape, dtype=_f32):
    return jax.ShapeDtypeStruct(shape, dtype)


def _sc_att_z(srcs, dsts, stab, heads, sd_off, with_cnt):
    @functools.partial(
        pl.kernel,
        out_type=_sds((NC, N, LW)),
        mesh=_mesh,
        compiler_params=_sc_params,
        scratch_types=[
            pltpu.VMEM((EPT,), jnp.int32),
            pltpu.VMEM((EPT,), jnp.int32),
            [pltpu.VMEM((C, LW), _f32)] * 3,
            [pltpu.VMEM((C, LW), _f32)] * 3,
            [pltpu.VMEM((C, LW), _f32)] * 3,
            pltpu.VMEM_SHARED((N, LW), _f32),
            [pltpu.SemaphoreType.DMA] * 3,
            [pltpu.SemaphoreType.DMA] * 3,
        ],
    )
    def k(srcs_h, dsts_h, stab_h, zeros_h, zparts_h,
          srcbuf, dstbuf, ssbuf, sdbuf, pchunk, z_sh,
          gsem, ssem):
        c = lax.axis_index("c")
        s = lax.axis_index("s")
        w = s * NC + c
        pltpu.sync_copy(srcs_h.at[w], srcbuf)
        pltpu.sync_copy(dsts_h.at[w], dstbuf)

        @pl.when(s < WTN)
        def _zero():
            pltpu.sync_copy(zeros_h, z_sh.at[pl.ds(s * SEG, SEG)])

        plsc.subcore_barrier()
        iota = lax.iota(jnp.int32, 16)
        zero16 = jnp.zeros((16,), _f32)
        one16 = jnp.ones((16,), _f32)
        for b in range(3):
            for i in range(C):
                pchunk[b][i, pl.ds(0, 16)] = zero16

        def issue_g(q, b):
            srcv = srcbuf[pl.ds(q * C, C)]
            dstv = dstbuf[pl.ds(q * C, C)]
            pltpu.async_copy(stab_h.at[srcv], ssbuf[b], gsem[b])
            pltpu.async_copy(stab_h.at[dstv], sdbuf[b], gsem[b])

        def wait_g(b):
            iv = srcbuf[pl.ds(0, C)]
            pltpu.make_async_copy(stab_h.at[iv], ssbuf[b], gsem[b]).wait()
            pltpu.make_async_copy(stab_h.at[iv], sdbuf[b], gsem[b]).wait()

        def wait_s(b):
            iv = dstbuf[pl.ds(0, C)]
            pltpu.make_async_copy(pchunk[b], z_sh.at[iv], ssem[b]).wait()

        def compute(q, b):
            dstv = dstbuf[pl.ds(q * C, C)]
            for h in range(heads):
                sv = (plsc.load_gather(ssbuf[b], [iota, jnp.full((16,), h, jnp.int32)])
                      + plsc.load_gather(sdbuf[b], [iota, jnp.full((16,), sd_off + h, jnp.int32)]))
                ev = jnp.where(sv > 0, sv, 0.2 * sv)
                pv = jnp.exp(ev)
                plsc.store_scatter(pchunk[b], [iota, jnp.full((16,), h, jnp.int32)], pv)
            if with_cnt:
                for i in range(C):
                    pchunk[b][i, pl.ds(CNT0, 16)] = zero16
                bsrc = plsc.load_gather(ssbuf[b], [iota, jnp.full((16,), BCOL, jnp.int32)])
                plsc.store_scatter(pchunk[b], [iota, CNT0 + bsrc.astype(jnp.int32)], one16)
            pltpu.async_copy(pchunk[b], z_sh.at[dstv], ssem[b], add=True)

        D = 3
        for i in range(D - 1):
            issue_g(i, i)
        nmain = (NCH - (D - 1)) // D

        def group(jj, carry):
            for b in range(D):
                q = jj * D + b
                issue_g(q + D - 1, (b + D - 1) % D)
                wait_g(b)

                @pl.when(q >= D)
                def _():
                    wait_s(b)

                compute(q, b)
            return carry

        lax.fori_loop(0, nmain, group, 0)
        for tq in range(D * nmain, NCH):
            b = tq % D
            if tq + D - 1 < NCH:
                issue_g(tq + D - 1, (b + D - 1) % D)
            wait_g(b)
            wait_s(b)
            compute(tq, b)
        for i in range(D):
            wait_s((NCH - D + i) % D)
        plsc.subcore_barrier()

        @pl.when(s < WTN)
        def _wb():
            pltpu.sync_copy(z_sh.at[pl.ds(s * SEG, SEG)],
                            zparts_h.at[c, pl.ds(s * SEG, SEG)])

    zeros = jnp.zeros((SEG, LW), _f32)
    return k(srcs, dsts, stab, zeros)


def _sc_att_agg(srcs, dsts, hext, rw, heads, oc):
    hw = heads * oc
    hwx = hw + LW
    D = 2 if hw >= 1024 else 3
    nseg = 25
    segc = NCH // nseg
    sege = segc * C

    @functools.partial(
        pl.kernel,
        out_type=_sds((NC, N, LW)),
        mesh=_mesh,
        compiler_params=_sc_params,
        scratch_types=[
            pltpu.VMEM((1, sege), jnp.int32),
            pltpu.VMEM((1, sege), jnp.int32),
            [pltpu.VMEM((C, LW), _f32)] * D,
            [pltpu.VMEM((C, hwx), _f32)] * D,
            [pltpu.VMEM((C, LW), _f32)] * D,
            pltpu.VMEM_SHARED((N, LW), _f32),
            [pltpu.SemaphoreType.DMA] * D,
            [pltpu.SemaphoreType.DMA] * D,
        ],
    )
    def k(srcs_h, dsts_h, hext_h, rw_h, zeros_h, oparts_h,
          srcbuf, dstbuf, rbuf, hbuf, msgbuf, o_sh,
          gsem, ssem):
        c = lax.axis_index("c")
        s = lax.axis_index("s")
        w = s * NC + c

        @pl.when(s < WTN)
        def _zero():
            pltpu.sync_copy(zeros_h, o_sh.at[pl.ds(s * SEG, SEG)])

        plsc.subcore_barrier()
        iota = lax.iota(jnp.int32, 16)

        def issue_g(q, b):
            srcv = srcbuf[0, pl.ds(q * C, C)]
            dstv = dstbuf[0, pl.ds(q * C, C)]
            pltpu.async_copy(rw_h.at[dstv], rbuf[b], gsem[b])
            pltpu.async_copy(hext_h.at[srcv], hbuf[b], gsem[b])

        def wait_g(b):
            iv = srcbuf[0, pl.ds(0, C)]
            pltpu.make_async_copy(rw_h.at[iv], rbuf[b], gsem[b]).wait()
            pltpu.make_async_copy(hext_h.at[iv], hbuf[b], gsem[b]).wait()

        def wait_s(b):
            iv = dstbuf[0, pl.ds(0, C)]
            pltpu.make_async_copy(msgbuf[b], o_sh.at[iv], ssem[b]).wait()

        def compute(q, b):
            dstv = dstbuf[0, pl.ds(q * C, C)]
            wv = []
            for h in range(heads):
                ssv = plsc.load_gather(hbuf[b], [iota, jnp.full((16,), hw + h, jnp.int32)])
                sdv = plsc.load_gather(rbuf[b], [iota, jnp.full((16,), 2 * heads + h, jnp.int32)])
                ev = ssv + sdv
                pv = jnp.exp(jnp.where(ev > 0, ev, 0.2 * ev))
                rv = plsc.load_gather(rbuf[b], [iota, jnp.full((16,), h, jnp.int32)])
                wv.append(pv * rv)

            def edge(ep, ecarry):
                for u in range(2):
                    e = ep * 2 + u
                    eidx = jnp.full((16,), e, jnp.int32)
                    ws = [wv[h].at[eidx].get(mode="promise_in_bounds")
                          for h in range(heads)]
                    for kk in range(oc // 16):
                        acc = ws[0] * hbuf[b][e, pl.ds(kk * 16, 16)]
                        for h in range(1, heads):
                            acc = acc + ws[h] * hbuf[b][e, pl.ds(h * oc + kk * 16, 16)]
                        msgbuf[b][e, pl.ds(kk * 16, 16)] = acc
                return ecarry

            lax.fori_loop(0, C // 2, edge, 0)
            pltpu.async_copy(msgbuf[b], o_sh.at[dstv], ssem[b], add=True)

        def segment(sg, carry):
            pltpu.sync_copy(srcs_h.at[w, sg], srcbuf)
            pltpu.sync_copy(dsts_h.at[w, sg], dstbuf)
            for i in range(D - 1):
                issue_g(i, i)
            nmain = (segc - (D - 1)) // D

            def group(jj, pcarry):
                for b in range(D):
                    q = jj * D + b
                    issue_g(q + D - 1, (b + D - 1) % D)
                    wait_g(b)

                    @pl.when(q >= D)
                    def _():
                        wait_s(b)

                    compute(q, b)
                return pcarry

            lax.fori_loop(0, nmain, group, 0)
            for tq in range(D * nmain, segc):
                b = tq % D
                if tq + D - 1 < segc:
                    issue_g(tq + D - 1, (b + D - 1) % D)
                wait_g(b)
                wait_s(b)
                compute(tq, b)
            for i in range(D):
                wait_s((segc - D + i) % D)
            return carry

        lax.fori_loop(0, nseg, segment, 0)
        plsc.subcore_barrier()

        @pl.when(s < WTN)
        def _wb():
            pltpu.sync_copy(o_sh.at[pl.ds(s * SEG, SEG)],
                            oparts_h.at[c, pl.ds(s * SEG, SEG)])

    zeros = jnp.zeros((SEG, LW), _f32)
    return k(srcs.reshape(NW, nseg, 1, sege), dsts.reshape(NW, nseg, 1, sege),
             hext, rw, zeros)


_BLK = 1000


def _full(shape):
    return pl.BlockSpec(shape, lambda j: (0,) * len(shape))


def _rows(cols):
    return pl.BlockSpec((_BLK, cols), lambda j: (j, 0))


def _tc_mm_proj(x, W, A, batch2d):
    din = x.shape[1]
    dh = W.shape[1]

    def body(x_r, w_r, a_r, bt_r, h_r, s_r):
        h = jnp.dot(x_r[...], w_r[...], preferred_element_type=_f32)
        ss = jnp.dot(h, a_r[...], preferred_element_type=_f32)
        iot = lax.broadcasted_iota(jnp.int32, (1, LW), 1)
        bt = bt_r[...].astype(_f32)
        sv = jnp.where(iot == BCOL, bt, ss)
        h_r[...] = jnp.concatenate([h, sv], axis=1)
        s_r[...] = sv

    return pl.pallas_call(
        body,
        grid=(N // _BLK,),
        in_specs=[_rows(din), _full((din, dh)), _full((dh, LW)),
                  pl.BlockSpec((_BLK, 1), lambda j: (j, 0))],
        out_specs=[_rows(dh + LW), _rows(LW)],
        out_shape=[_sds((N, dh + LW)), _sds((N, LW))],
    )(x, W, A, batch2d)


def _tc_rinv(z0, z1, S, heads):
    def body(z0_r, z1_r, s_r, rw_r):
        ss = s_r[:, :heads]
        sd = s_r[:, heads:2 * heads]
        e = ss + sd
        p = jnp.exp(jnp.where(e > 0, e, 0.2 * e))
        zt = z0_r[:, :heads] + z1_r[:, :heads] + p + 1e-16
        r = 1.0 / (zt * heads)
        pad = jnp.zeros((_BLK, LW - 3 * heads), _f32)
        rw_r[...] = jnp.concatenate([r, p * r, sd, pad], axis=1)

    return pl.pallas_call(
        body,
        grid=(N // _BLK,),
        in_specs=[_rows(LW), _rows(LW), _rows(LW)],
        out_specs=_rows(LW),
        out_shape=_sds((N, LW)),
    )(z0, z1, S)


def _tc_post_pre(o0, o1, rw, hprev, b, Wn, An, heads, oc, relu, batch2d):
    dn = Wn.shape[1]
    hw = heads * oc

    def body(o0_r, o1_r, rw_r, hp_r, b_r, w_r, a_r, bt_r, h_r, s_r):
        o = o0_r[:, :oc] + o1_r[:, :oc]
        hp = hp_r[...]
        wl = rw_r[:, heads:2 * heads]
        for h in range(heads):
            o = o + wl[:, h:h + 1] * hp[:, h * oc:(h + 1) * oc]
        x1 = o + b_r[...]
        if relu:
            x1 = jnp.maximum(x1, 0.0)
        hh = jnp.dot(x1, w_r[...], preferred_element_type=_f32)
        ss = jnp.dot(hh, a_r[...], preferred_element_type=_f32)
        iot = lax.broadcasted_iota(jnp.int32, (1, LW), 1)
        bt = bt_r[...].astype(_f32)
        sv = jnp.where(iot == BCOL, bt, ss)
        h_r[...] = jnp.concatenate([hh, sv], axis=1)
        s_r[...] = sv

    return pl.pallas_call(
        body,
        grid=(N // _BLK,),
        in_specs=[_rows(LW), _rows(LW), _rows(LW), _rows(hw + LW),
                  _full((1, oc)), _full((oc, dn)), _full((dn, LW)),
                  pl.BlockSpec((_BLK, 1), lambda j: (j, 0))],
        out_specs=[_rows(dn + LW), _rows(LW)],
        out_shape=[_sds((N, dn + LW)), _sds((N, LW))],
    )(o0, o1, rw, hprev, b, Wn, An, batch2d)


def _tc_pool(o0, o1, rw, hprev, b, batch2d, Wg1, bg1, Wg2, bg2, W_d0, avs, avd,
             heads, oc):
    ng = N // _BLK
    hw = heads * oc
    d0 = W_d0.shape[1]

    def body(o0_r, o1_r, rw_r, hp_r, b_r, bt_r, wg1_r, bg1_r, wg2_r, bg2_r,
             wd0_r, avs_r, avd_r, t_r, tsrow_r, tscol_r, tdcol_r, sg, zg):
        j = pl.program_id(0)
        o = o0_r[:, :oc] + o1_r[:, :oc]
        hp = hp_r[...]
        wl = rw_r[:, heads:2 * heads]
        for h in range(heads):
            o = o + wl[:, h:h + 1] * hp[:, h * oc:(h + 1) * oc]
        hn = o + b_r[...]
        g = jnp.maximum(jnp.dot(hn, wg1_r[...], preferred_element_type=_f32)
                        + bg1_r[...], 0.0)
        g = jnp.dot(g, wg2_r[...], preferred_element_type=_f32) + bg2_r[...]
        p = jnp.exp(g)
        iot = lax.broadcasted_iota(jnp.int32, (1, G), 1)
        mask = (bt_r[...] == iot).astype(_f32)
        mp = mask * p

        @pl.when(j == 0)
        def _():
            sg[...] = jnp.zeros_like(sg)
            zg[...] = jnp.zeros_like(zg)

        dnm = (((0,), (0,)), ((), ()))
        sg[...] += lax.dot_general(mp, hn, dnm, preferred_element_type=_f32)
        zg[...] += lax.dot_general(mp, jnp.ones((_BLK, 1), _f32), dnm,
                                   preferred_element_type=_f32)
        pooled = sg[...] / (zg[...] + 1e-16)
        t = jnp.dot(pooled, wd0_r[...], preferred_element_type=_f32)
        t_r[...] = t
        tscol_r[...] = jnp.dot(t, avs_r[...], preferred_element_type=_f32)
        tdcol_r[...] = jnp.dot(t, avd_r[...], preferred_element_type=_f32)
        tsrow_r[...] = lax.dot_general(avs_r[...], t, (((0,), (1,)), ((), ())),
                                       preferred_element_type=_f32)

    return pl.pallas_call(
        body,
        grid=(ng,),
        in_specs=[_rows(LW), _rows(LW), _rows(LW), _rows(hw + LW),
                  _full((1, oc)), pl.BlockSpec((_BLK, 1), lambda j: (j, 0)),
                  _full(Wg1.shape), _full((1, oc)), _full(Wg2.shape),
                  _full((1, 1)), _full(W_d0.shape), _full((d0, 1)),
                  _full((d0, 1))],
        out_specs=[_full((G, d0)), _full((1, G)), _full((G, 1)), _full((G, 1))],
        out_shape=[_sds((G, d0)), _sds((1, G)), _sds((G, 1)), _sds((G, 1))],
        scratch_shapes=[pltpu.VMEM((G, oc), _f32), pltpu.VMEM((G, 1), _f32)],
    )(o0, o1, rw, hprev, b, batch2d, Wg1, bg1, Wg2, bg2, W_d0, avs, avd)


def _tc_d0(zp0, zp1, batch2d, T, tsrow, tscol, tdcol, b, Wn, An):
    d0 = T.shape[1]
    dn = Wn.shape[1]

    def body(z0_r, z1_r, bt_r, t_r, tsr_r, tsc_r, tdc_r, b_r, w_r, a_r,
             h_r, s_r):
        iot = lax.broadcasted_iota(jnp.int32, (1, G), 1)
        onehot = (bt_r[...] == iot).astype(_f32)
        cnt = z0_r[:, CNT0:CNT0 + G] + z1_r[:, CNT0:CNT0 + G]
        td_n = jnp.dot(onehot, tdc_r[...], preferred_element_type=_f32)
        ts_n = jnp.dot(onehot, tsc_r[...], preferred_element_type=_f32)
        em = tsr_r[...] + td_n
        m = jnp.exp(jnp.where(em > 0, em, 0.2 * em))
        ed = ts_n + td_n
        pd = jnp.exp(jnp.where(ed > 0, ed, 0.2 * ed))
        cm = cnt * m + onehot * pd
        z = jnp.dot(cm, jnp.ones((G, 1), _f32), preferred_element_type=_f32)
        o = jnp.dot(cm, t_r[...], preferred_element_type=_f32) / (z + 1e-16)
        h2 = jnp.maximum(o + b_r[...], 0.0)
        hh = jnp.dot(h2, w_r[...], preferred_element_type=_f32)
        sv = jnp.dot(hh, a_r[...], preferred_element_type=_f32)
        h_r[...] = jnp.concatenate([hh, sv], axis=1)
        s_r[...] = sv

    return pl.pallas_call(
        body,
        grid=(N // _BLK,),
        in_specs=[_rows(LW), _rows(LW), pl.BlockSpec((_BLK, 1), lambda j: (j, 0)),
                  _full((G, d0)), _full((1, G)), _full((G, 1)), _full((G, 1)),
                  _full((1, d0)), _full((d0, dn)), _full((dn, LW))],
        out_specs=[_rows(dn + LW), _rows(LW)],
        out_shape=[_sds((N, dn + LW)), _sds((N, LW))],
    )(zp0, zp1, batch2d, T, tsrow, tscol, tdcol, b, Wn, An)


def _tc_final(o0, o1, rw, hmat, b, oc):
    def body(o0_r, o1_r, rw_r, h_r, b_r, out_r):
        wl = rw_r[:, 1:2]
        out_r[...] = o0_r[:, :oc] + o1_r[:, :oc] + wl * h_r[:, :oc] + b_r[...]

    return pl.pallas_call(
        body,
        grid=(N // _BLK,),
        in_specs=[_rows(LW), _rows(LW), _rows(LW), _rows(oc + LW), _full((1, oc))],
        out_specs=_rows(oc),
        out_shape=_sds((N, oc)),
    )(o0, o1, rw, hmat, b)


def _blockdiag(a_s, a_d):
    heads, oc = a_s.shape[1], a_s.shape[2]
    eye = jnp.eye(heads, dtype=_f32)
    bs = (a_s[0][:, :, None] * eye[:, None, :]).reshape(heads * oc, heads)
    bd = (a_d[0][:, :, None] * eye[:, None, :]).reshape(heads * oc, heads)
    pad = jnp.zeros((heads * oc, LW - 2 * heads), _f32)
    return jnp.concatenate([bs, bd, pad], axis=1)


def kernel(x, edge_index, batch, W_e0, a_src_e0, a_dst_e0, b_e0, W_e1,
           a_src_e1, a_dst_e1, b_e1, Wg1, bg1, Wg2, bg2, W_d0, a_src_d0,
           a_dst_d0, b_d0, W_d1, a_src_d1, a_dst_d1, b_d1):
    src = edge_index[0].astype(jnp.int32).reshape(NW, EPT)
    dst = edge_index[1].astype(jnp.int32).reshape(NW, EPT)
    batch2d = batch.astype(jnp.int32).reshape(N, 1)

    A0 = _blockdiag(a_src_e0, a_dst_e0)
    A1 = _blockdiag(a_src_e1, a_dst_e1)
    A3 = _blockdiag(a_src_d1, a_dst_d1)
    avs_d0 = a_src_d0[0, 0, :].reshape(-1, 1)
    avd_d0 = a_dst_d0[0, 0, :].reshape(-1, 1)

    H0, S0 = _tc_mm_proj(x, W_e0, A0, batch2d)
    zp0 = _sc_att_z(src, dst, S0, 8, 8, with_cnt=True)
    RW0 = _tc_rinv(zp0[0], zp0[1], S0, 8)
    op0 = _sc_att_agg(src, dst, H0, RW0, 8, 128)
    H1, S1 = _tc_post_pre(op0[0], op0[1], RW0, H0, b_e0.reshape(1, -1),
                          W_e1, A1, 8, 128, True, batch2d)

    zp1 = _sc_att_z(src, dst, S1, 8, 8, with_cnt=False)
    RW1 = _tc_rinv(zp1[0], zp1[1], S1, 8)
    op1 = _sc_att_agg(src, dst, H1, RW1, 8, 64)

    T, tsrow, tscol, tdcol = _tc_pool(
        op1[0], op1[1], RW1, H1, b_e1.reshape(1, -1), batch2d, Wg1,
        bg1.reshape(1, -1), Wg2, bg2.reshape(1, -1), W_d0, avs_d0, avd_d0,
        8, 64)
    H3, S3 = _tc_d0(zp0[0], zp0[1], batch2d, T, tsrow, tscol, tdcol,
                    b_d0.reshape(1, -1), W_d1, A3)

    zp3 = _sc_att_z(src, dst, S3, 1, 1, with_cnt=False)
    RW3 = _tc_rinv(zp3[0], zp3[1], S3, 1)
    op3 = _sc_att_agg(src, dst, H3, RW3, 1, 128)
    out = _tc_final(op3[0], op3[1], RW3, H3, b_d1.reshape(1, -1), 128)
    return out


# speedup vs baseline: 31.1067x; 1.0080x over previous
import functools

import jax
import jax.numpy as jnp
from jax import lax
from jax.experimental import pallas as pl
from jax.experimental.pallas import tpu as pltpu
from jax.experimental.pallas import tpu_sc as plsc

N = 10000
E = 320000
G = 16
NC = 2
NS = 16
NW = NC * NS
EPT = E // NW
C = 16
NCH = EPT // C
WTN = 10
SEG = N // WTN
LW = 128
BCOL = 120
CNT0 = 16

_mesh = plsc.VectorSubcoreMesh(
    core_axis_name="c", subcore_axis_name="s", num_cores=NC, num_subcores=NS)
_sc_params = pltpu.CompilerParams(needs_layout_passes=False)

_f32 = jnp.float32


def _sds(shape, dtype=_f32):
    return jax.ShapeDtypeStruct(shape, dtype)


def _sc_att_z(srcs, dsts, stab, heads, sd_off, with_cnt):
    @functools.partial(
        pl.kernel,
        out_type=_sds((NC, N, LW)),
        mesh=_mesh,
        compiler_params=_sc_params,
        scratch_types=[
            pltpu.VMEM((EPT,), jnp.int32),
            pltpu.VMEM((EPT,), jnp.int32),
            [pltpu.VMEM((C, LW), _f32)] * 3,
            [pltpu.VMEM((C, LW), _f32)] * 3,
            [pltpu.VMEM((C, LW), _f32)] * 3,
            pltpu.VMEM_SHARED((N, LW), _f32),
            [pltpu.SemaphoreType.DMA] * 3,
            [pltpu.SemaphoreType.DMA] * 3,
        ],
    )
    def k(srcs_h, dsts_h, stab_h, zeros_h, zparts_h,
          srcbuf, dstbuf, ssbuf, sdbuf, pchunk, z_sh,
          gsem, ssem):
        c = lax.axis_index("c")
        s = lax.axis_index("s")
        w = s * NC + c
        pltpu.sync_copy(srcs_h.at[w], srcbuf)
        pltpu.sync_copy(dsts_h.at[w], dstbuf)

        @pl.when(s < WTN)
        def _zero():
            pltpu.sync_copy(zeros_h, z_sh.at[pl.ds(s * SEG, SEG)])

        plsc.subcore_barrier()
        iota = lax.iota(jnp.int32, 16)
        zero16 = jnp.zeros((16,), _f32)
        one16 = jnp.ones((16,), _f32)
        for b in range(3):
            for i in range(C):
                pchunk[b][i, pl.ds(0, 16)] = zero16

        def issue_g(q, b):
            srcv = srcbuf[pl.ds(q * C, C)]
            dstv = dstbuf[pl.ds(q * C, C)]
            pltpu.async_copy(stab_h.at[srcv], ssbuf[b], gsem[b])
            pltpu.async_copy(stab_h.at[dstv], sdbuf[b], gsem[b])

        def wait_g(b):
            iv = srcbuf[pl.ds(0, C)]
            pltpu.make_async_copy(stab_h.at[iv], ssbuf[b], gsem[b]).wait()
            pltpu.make_async_copy(stab_h.at[iv], sdbuf[b], gsem[b]).wait()

        def wait_s(b):
            iv = dstbuf[pl.ds(0, C)]
            pltpu.make_async_copy(pchunk[b], z_sh.at[iv], ssem[b]).wait()

        def compute(q, b):
            dstv = dstbuf[pl.ds(q * C, C)]
            for h in range(heads):
                sv = (plsc.load_gather(ssbuf[b], [iota, jnp.full((16,), h, jnp.int32)])
                      + plsc.load_gather(sdbuf[b], [iota, jnp.full((16,), sd_off + h, jnp.int32)]))
                ev = jnp.where(sv > 0, sv, 0.2 * sv)
                pv = jnp.exp(ev)
                plsc.store_scatter(pchunk[b], [iota, jnp.full((16,), h, jnp.int32)], pv)
            if with_cnt:
                for i in range(C):
                    pchunk[b][i, pl.ds(CNT0, 16)] = zero16
                bsrc = plsc.load_gather(ssbuf[b], [iota, jnp.full((16,), BCOL, jnp.int32)])
                plsc.store_scatter(pchunk[b], [iota, CNT0 + bsrc.astype(jnp.int32)], one16)
            pltpu.async_copy(pchunk[b], z_sh.at[dstv], ssem[b], add=True)

        D = 3
        for i in range(D - 1):
            issue_g(i, i)
        nmain = (NCH - (D - 1)) // D

        def group(jj, carry):
            for b in range(D):
                q = jj * D + b
                issue_g(q + D - 1, (b + D - 1) % D)
                wait_g(b)

                @pl.when(q >= D)
                def _():
                    wait_s(b)

                compute(q, b)
            return carry

        lax.fori_loop(0, nmain, group, 0)
        for tq in range(D * nmain, NCH):
            b = tq % D
            if tq + D - 1 < NCH:
                issue_g(tq + D - 1, (b + D - 1) % D)
            wait_g(b)
            wait_s(b)
            compute(tq, b)
        for i in range(D):
            wait_s((NCH - D + i) % D)
        plsc.subcore_barrier()

        @pl.when(s < WTN)
        def _wb():
            pltpu.sync_copy(z_sh.at[pl.ds(s * SEG, SEG)],
                            zparts_h.at[c, pl.ds(s * SEG, SEG)])

    zeros = jnp.zeros((SEG, LW), _f32)
    return k(srcs, dsts, stab, zeros)


def _sc_att_agg(srcs, dsts, hext, rw, heads, oc):
    hw = heads * oc
    hwx = hw + LW
    D = 2 if hw >= 1024 else 3
    nseg = 25
    segc = NCH // nseg
    sege = segc * C

    @functools.partial(
        pl.kernel,
        out_type=_sds((NC, N, LW)),
        mesh=_mesh,
        compiler_params=_sc_params,
        scratch_types=[
            pltpu.VMEM((1, sege), jnp.int32),
            pltpu.VMEM((1, sege), jnp.int32),
            [pltpu.VMEM((C, LW), _f32)] * D,
            [pltpu.VMEM((C, hwx), _f32)] * D,
            [pltpu.VMEM((C, LW), _f32)] * D,
            pltpu.VMEM_SHARED((N, LW), _f32),
            [pltpu.SemaphoreType.DMA] * D,
            [pltpu.SemaphoreType.DMA] * D,
        ],
    )
    def k(srcs_h, dsts_h, hext_h, rw_h, zeros_h, oparts_h,
          srcbuf, dstbuf, rbuf, hbuf, msgbuf, o_sh,
          gsem, ssem):
        c = lax.axis_index("c")
        s = lax.axis_index("s")
        w = s * NC + c

        @pl.when(s < WTN)
        def _zero():
            pltpu.sync_copy(zeros_h, o_sh.at[pl.ds(s * SEG, SEG)])

        plsc.subcore_barrier()
        iota = lax.iota(jnp.int32, 16)

        def issue_g(q, b):
            srcv = srcbuf[0, pl.ds(q * C, C)]
            dstv = dstbuf[0, pl.ds(q * C, C)]
            pltpu.async_copy(rw_h.at[dstv], rbuf[b], gsem[b])
            pltpu.async_copy(hext_h.at[srcv], hbuf[b], gsem[b])

        def wait_g(b):
            iv = srcbuf[0, pl.ds(0, C)]
            pltpu.make_async_copy(rw_h.at[iv], rbuf[b], gsem[b]).wait()
            pltpu.make_async_copy(hext_h.at[iv], hbuf[b], gsem[b]).wait()

        def wait_s(b):
            iv = dstbuf[0, pl.ds(0, C)]
            pltpu.make_async_copy(msgbuf[b], o_sh.at[iv], ssem[b]).wait()

        def compute(q, b):
            dstv = dstbuf[0, pl.ds(q * C, C)]
            wv = []
            for h in range(heads):
                ssv = plsc.load_gather(hbuf[b], [iota, jnp.full((16,), hw + h, jnp.int32)])
                sdv = plsc.load_gather(rbuf[b], [iota, jnp.full((16,), 2 * heads + h, jnp.int32)])
                ev = ssv + sdv
                pv = jnp.exp(jnp.where(ev > 0, ev, 0.2 * ev))
                rv = plsc.load_gather(rbuf[b], [iota, jnp.full((16,), h, jnp.int32)])
                wv.append(pv * rv)

            def edge(e, ecarry):
                eidx = jnp.full((16,), e, jnp.int32)
                ws = [wv[h].at[eidx].get(mode="promise_in_bounds")
                      for h in range(heads)]
                for kk in range(oc // 16):
                    acc = ws[0] * hbuf[b][e, pl.ds(kk * 16, 16)]
                    for h in range(1, heads):
                        acc = acc + ws[h] * hbuf[b][e, pl.ds(h * oc + kk * 16, 16)]
                    msgbuf[b][e, pl.ds(kk * 16, 16)] = acc
                return ecarry

            lax.fori_loop(0, C, edge, 0)
            pltpu.async_copy(msgbuf[b], o_sh.at[dstv], ssem[b], add=True)

        def segment(sg, carry):
            pltpu.sync_copy(srcs_h.at[w, sg], srcbuf)
            pltpu.sync_copy(dsts_h.at[w, sg], dstbuf)
            for i in range(D - 1):
                issue_g(i, i)
            nmain = (segc - (D - 1)) // D

            def group(jj, pcarry):
                for b in range(D):
                    q = jj * D + b
                    issue_g(q + D - 1, (b + D - 1) % D)
                    wait_g(b)

                    @pl.when(q >= D)
                    def _():
                        wait_s(b)

                    compute(q, b)
                return pcarry

            lax.fori_loop(0, nmain, group, 0)
            for tq in range(D * nmain, segc):
                b = tq % D
                if tq + D - 1 < segc:
                    issue_g(tq + D - 1, (b + D - 1) % D)
                wait_g(b)
                wait_s(b)
                compute(tq, b)
            for i in range(D):
                wait_s((segc - D + i) % D)
            return carry

        lax.fori_loop(0, nseg, segment, 0)
        plsc.subcore_barrier()

        @pl.when(s < WTN)
        def _wb():
            pltpu.sync_copy(o_sh.at[pl.ds(s * SEG, SEG)],
                            oparts_h.at[c, pl.ds(s * SEG, SEG)])

    zeros = jnp.zeros((SEG, LW), _f32)
    return k(srcs.reshape(NW, nseg, 1, sege), dsts.reshape(NW, nseg, 1, sege),
             hext, rw, zeros)


_BLK = 1000


def _full(shape):
    return pl.BlockSpec(shape, lambda j: (0,) * len(shape))


def _rows(cols):
    return pl.BlockSpec((_BLK, cols), lambda j: (j, 0))


def _tc_mm_proj(x, W, A, batch2d):
    din = x.shape[1]
    dh = W.shape[1]

    def body(x_r, w_r, a_r, bt_r, h_r, s_r):
        h = jnp.dot(x_r[...], w_r[...], preferred_element_type=_f32)
        ss = jnp.dot(h, a_r[...], preferred_element_type=_f32)
        iot = lax.broadcasted_iota(jnp.int32, (1, LW), 1)
        bt = bt_r[...].astype(_f32)
        sv = jnp.where(iot == BCOL, bt, ss)
        h_r[...] = jnp.concatenate([h, sv], axis=1)
        s_r[...] = sv

    return pl.pallas_call(
        body,
        grid=(N // _BLK,),
        in_specs=[_rows(din), _full((din, dh)), _full((dh, LW)),
                  pl.BlockSpec((_BLK, 1), lambda j: (j, 0))],
        out_specs=[_rows(dh + LW), _rows(LW)],
        out_shape=[_sds((N, dh + LW)), _sds((N, LW))],
    )(x, W, A, batch2d)


def _tc_rinv(z0, z1, S, heads):
    def body(z0_r, z1_r, s_r, rw_r):
        ss = s_r[:, :heads]
        sd = s_r[:, heads:2 * heads]
        e = ss + sd
        p = jnp.exp(jnp.where(e > 0, e, 0.2 * e))
        zt = z0_r[:, :heads] + z1_r[:, :heads] + p + 1e-16
        r = 1.0 / (zt * heads)
        pad = jnp.zeros((_BLK, LW - 3 * heads), _f32)
        rw_r[...] = jnp.concatenate([r, p * r, sd, pad], axis=1)

    return pl.pallas_call(
        body,
        grid=(N // _BLK,),
        in_specs=[_rows(LW), _rows(LW), _rows(LW)],
        out_specs=_rows(LW),
        out_shape=_sds((N, LW)),
    )(z0, z1, S)


def _tc_post_pre(o0, o1, rw, hprev, b, Wn, An, heads, oc, relu, batch2d):
    dn = Wn.shape[1]
    hw = heads * oc

    def body(o0_r, o1_r, rw_r, hp_r, b_r, w_r, a_r, bt_r, h_r, s_r):
        o = o0_r[:, :oc] + o1_r[:, :oc]
        hp = hp_r[...]
        wl = rw_r[:, heads:2 * heads]
        for h in range(heads):
            o = o + wl[:, h:h + 1] * hp[:, h * oc:(h + 1) * oc]
        x1 = o + b_r[...]
        if relu:
            x1 = jnp.maximum(x1, 0.0)
        hh = jnp.dot(x1, w_r[...], preferred_element_type=_f32)
        ss = jnp.dot(hh, a_r[...], preferred_element_type=_f32)
        iot = lax.broadcasted_iota(jnp.int32, (1, LW), 1)
        bt = bt_r[...].astype(_f32)
        sv = jnp.where(iot == BCOL, bt, ss)
        h_r[...] = jnp.concatenate([hh, sv], axis=1)
        s_r[...] = sv

    return pl.pallas_call(
        body,
        grid=(N // _BLK,),
        in_specs=[_rows(LW), _rows(LW), _rows(LW), _rows(hw + LW),
                  _full((1, oc)), _full((oc, dn)), _full((dn, LW)),
                  pl.BlockSpec((_BLK, 1), lambda j: (j, 0))],
        out_specs=[_rows(dn + LW), _rows(LW)],
        out_shape=[_sds((N, dn + LW)), _sds((N, LW))],
    )(o0, o1, rw, hprev, b, Wn, An, batch2d)


def _tc_pool(o0, o1, rw, hprev, b, batch2d, Wg1, bg1, Wg2, bg2, W_d0, avs, avd,
             heads, oc):
    ng = N // _BLK
    hw = heads * oc
    d0 = W_d0.shape[1]

    def body(o0_r, o1_r, rw_r, hp_r, b_r, bt_r, wg1_r, bg1_r, wg2_r, bg2_r,
             wd0_r, avs_r, avd_r, t_r, tsrow_r, tscol_r, tdcol_r, sg, zg):
        j = pl.program_id(0)
        o = o0_r[:, :oc] + o1_r[:, :oc]
        hp = hp_r[...]
        wl = rw_r[:, heads:2 * heads]
        for h in range(heads):
            o = o + wl[:, h:h + 1] * hp[:, h * oc:(h + 1) * oc]
        hn = o + b_r[...]
        g = jnp.maximum(jnp.dot(hn, wg1_r[...], preferred_element_type=_f32)
                        + bg1_r[...], 0.0)
        g = jnp.dot(g, wg2_r[...], preferred_element_type=_f32) + bg2_r[...]
        p = jnp.exp(g)
        iot = lax.broadcasted_iota(jnp.int32, (1, G), 1)
        mask = (bt_r[...] == iot).astype(_f32)
        mp = mask * p

        @pl.when(j == 0)
        def _():
            sg[...] = jnp.zeros_like(sg)
            zg[...] = jnp.zeros_like(zg)

        dnm = (((0,), (0,)), ((), ()))
        sg[...] += lax.dot_general(mp, hn, dnm, preferred_element_type=_f32)
        zg[...] += lax.dot_general(mp, jnp.ones((_BLK, 1), _f32), dnm,
                                   preferred_element_type=_f32)
        pooled = sg[...] / (zg[...] + 1e-16)
        t = jnp.dot(pooled, wd0_r[...], preferred_element_type=_f32)
        t_r[...] = t
        tscol_r[...] = jnp.dot(t, avs_r[...], preferred_element_type=_f32)
        tdcol_r[...] = jnp.dot(t, avd_r[...], preferred_element_type=_f32)
        tsrow_r[...] = lax.dot_general(avs_r[...], t, (((0,), (1,)), ((), ())),
                                       preferred_element_type=_f32)

    return pl.pallas_call(
        body,
        grid=(ng,),
        in_specs=[_rows(LW), _rows(LW), _rows(LW), _rows(hw + LW),
                  _full((1, oc)), pl.BlockSpec((_BLK, 1), lambda j: (j, 0)),
                  _full(Wg1.shape), _full((1, oc)), _full(Wg2.shape),
                  _full((1, 1)), _full(W_d0.shape), _full((d0, 1)),
                  _full((d0, 1))],
        out_specs=[_full((G, d0)), _full((1, G)), _full((G, 1)), _full((G, 1))],
        out_shape=[_sds((G, d0)), _sds((1, G)), _sds((G, 1)), _sds((G, 1))],
        scratch_shapes=[pltpu.VMEM((G, oc), _f32), pltpu.VMEM((G, 1), _f32)],
    )(o0, o1, rw, hprev, b, batch2d, Wg1, bg1, Wg2, bg2, W_d0, avs, avd)


def _tc_d0(zp0, zp1, batch2d, T, tsrow, tscol, tdcol, b, Wn, An):
    d0 = T.shape[1]
    dn = Wn.shape[1]

    def body(z0_r, z1_r, bt_r, t_r, tsr_r, tsc_r, tdc_r, b_r, w_r, a_r,
             h_r, s_r):
        iot = lax.broadcasted_iota(jnp.int32, (1, G), 1)
        onehot = (bt_r[...] == iot).astype(_f32)
        cnt = z0_r[:, CNT0:CNT0 + G] + z1_r[:, CNT0:CNT0 + G]
        td_n = jnp.dot(onehot, tdc_r[...], preferred_element_type=_f32)
        ts_n = jnp.dot(onehot, tsc_r[...], preferred_element_type=_f32)
        em = tsr_r[...] + td_n
        m = jnp.exp(jnp.where(em > 0, em, 0.2 * em))
        ed = ts_n + td_n
        pd = jnp.exp(jnp.where(ed > 0, ed, 0.2 * ed))
        cm = cnt * m + onehot * pd
        z = jnp.dot(cm, jnp.ones((G, 1), _f32), preferred_element_type=_f32)
        o = jnp.dot(cm, t_r[...], preferred_element_type=_f32) / (z + 1e-16)
        h2 = jnp.maximum(o + b_r[...], 0.0)
        hh = jnp.dot(h2, w_r[...], preferred_element_type=_f32)
        sv = jnp.dot(hh, a_r[...], preferred_element_type=_f32)
        h_r[...] = jnp.concatenate([hh, sv], axis=1)
        s_r[...] = sv

    return pl.pallas_call(
        body,
        grid=(N // _BLK,),
        in_specs=[_rows(LW), _rows(LW), pl.BlockSpec((_BLK, 1), lambda j: (j, 0)),
                  _full((G, d0)), _full((1, G)), _full((G, 1)), _full((G, 1)),
                  _full((1, d0)), _full((d0, dn)), _full((dn, LW))],
        out_specs=[_rows(dn + LW), _rows(LW)],
        out_shape=[_sds((N, dn + LW)), _sds((N, LW))],
    )(zp0, zp1, batch2d, T, tsrow, tscol, tdcol, b, Wn, An)


def _tc_final(o0, o1, rw, hmat, b, oc):
    def body(o0_r, o1_r, rw_r, h_r, b_r, out_r):
        wl = rw_r[:, 1:2]
        out_r[...] = o0_r[:, :oc] + o1_r[:, :oc] + wl * h_r[:, :oc] + b_r[...]

    return pl.pallas_call(
        body,
        grid=(N // _BLK,),
        in_specs=[_rows(LW), _rows(LW), _rows(LW), _rows(oc + LW), _full((1, oc))],
        out_specs=_rows(oc),
        out_shape=_sds((N, oc)),
    )(o0, o1, rw, hmat, b)


def _blockdiag(a_s, a_d):
    heads, oc = a_s.shape[1], a_s.shape[2]
    eye = jnp.eye(heads, dtype=_f32)
    bs = (a_s[0][:, :, None] * eye[:, None, :]).reshape(heads * oc, heads)
    bd = (a_d[0][:, :, None] * eye[:, None, :]).reshape(heads * oc, heads)
    pad = jnp.zeros((heads * oc, LW - 2 * heads), _f32)
    return jnp.concatenate([bs, bd, pad], axis=1)


def kernel(x, edge_index, batch, W_e0, a_src_e0, a_dst_e0, b_e0, W_e1,
           a_src_e1, a_dst_e1, b_e1, Wg1, bg1, Wg2, bg2, W_d0, a_src_d0,
           a_dst_d0, b_d0, W_d1, a_src_d1, a_dst_d1, b_d1):
    src = edge_index[0].astype(jnp.int32).reshape(NW, EPT)
    dst = edge_index[1].astype(jnp.int32).reshape(NW, EPT)
    batch2d = batch.astype(jnp.int32).reshape(N, 1)

    A0 = _blockdiag(a_src_e0, a_dst_e0)
    A1 = _blockdiag(a_src_e1, a_dst_e1)
    A3 = _blockdiag(a_src_d1, a_dst_d1)
    avs_d0 = a_src_d0[0, 0, :].reshape(-1, 1)
    avd_d0 = a_dst_d0[0, 0, :].reshape(-1, 1)

    H0, S0 = _tc_mm_proj(x, W_e0, A0, batch2d)
    zp0 = _sc_att_z(src, dst, S0, 8, 8, with_cnt=True)
    RW0 = _tc_rinv(zp0[0], zp0[1], S0, 8)
    op0 = _sc_att_agg(src, dst, H0, RW0, 8, 128)
    H1, S1 = _tc_post_pre(op0[0], op0[1], RW0, H0, b_e0.reshape(1, -1),
                          W_e1, A1, 8, 128, True, batch2d)

    zp1 = _sc_att_z(src, dst, S1, 8, 8, with_cnt=False)
    RW1 = _tc_rinv(zp1[0], zp1[1], S1, 8)
    op1 = _sc_att_agg(src, dst, H1, RW1, 8, 64)

    T, tsrow, tscol, tdcol = _tc_pool(
        op1[0], op1[1], RW1, H1, b_e1.reshape(1, -1), batch2d, Wg1,
        bg1.reshape(1, -1), Wg2, bg2.reshape(1, -1), W_d0, avs_d0, avd_d0,
        8, 64)
    H3, S3 = _tc_d0(zp0[0], zp0[1], batch2d, T, tsrow, tscol, tdcol,
                    b_d0.reshape(1, -1), W_d1, A3)

    zp3 = _sc_att_z(src, dst, S3, 1, 1, with_cnt=False)
    RW3 = _tc_rinv(zp3[0], zp3[1], S3, 1)
    op3 = _sc_att_agg(src, dst, H3, RW3, 1, 128)
    out = _tc_final(op3[0], op3[1], RW3, H3, b_d1.reshape(1, -1), 128)
    return out
